# Initial kernel scaffold; baseline (speedup 1.0000x reference)
import jax
import jax.numpy as jnp
from jax.experimental import pallas as pl


def kernel(atom_list, bond_list, atom_degree_list, bond_degree_list, atom_mask, params):
    raise NotImplementedError("write your pallas kernel here")


# speedup vs baseline: 7.4790x; 7.4790x over previous
import functools

import jax
import jax.numpy as jnp
from jax.experimental import pallas as pl
from jax.experimental.pallas import tpu as pltpu

B, M, D = 256, 128, 8
FA, FB, FP = 64, 16, 64
RADIUS, T_STEPS = 3, 2
MD = M * D

_PREC = jax.lax.Precision.HIGHEST


def _dot(a, b):
    return jax.lax.dot(a, b, precision=_PREC, preferred_element_type=jnp.float32)


def _lrelu(x):
    return jnp.where(x >= 0, x, 0.01 * x)


def _elu(x):
    return jnp.where(x > 0, x, jnp.exp(jnp.minimum(x, 0.0)) - 1.0)


def _softmax(x, axis):
    m = jnp.max(x, axis=axis, keepdims=True)
    e = jnp.exp(x - m)
    return e / jnp.sum(e, axis=axis, keepdims=True)


def _gru(x, h, Wi_t, Wh_t, bi, bh):
    gi = _dot(x, Wi_t) + bi
    gh = _dot(h, Wh_t) + bh
    r = jax.nn.sigmoid(gi[:, :FP] + gh[:, :FP])
    z = jax.nn.sigmoid(gi[:, FP:2 * FP] + gh[:, FP:2 * FP])
    n = jnp.tanh(gi[:, 2 * FP:] + r * gh[:, 2 * FP:])
    return (1.0 - z) * n + z * h


def _body(atom_ref, bond_ref, adeg_ref, bdeg_ref, amask_ref,
          Wa_t, ba, WnA_t, WnB_t, bn,
          wal_a, wal_n, bal, Wt_t, bt, Wi_t, Wh_t, bi, bh,
          wma_m, wma_x, bma, Wmt_t, bmt, mWi_t, mWh_t, mbi, mbh,
          wo, bo,
          af_out_ref, pred_out_ref):
    atom = atom_ref[0]
    bond = bond_ref[0]
    adeg = adeg_ref[0]
    bdeg = bdeg_ref[0]
    amask = amask_ref[0]

    col = jax.lax.broadcasted_iota(jnp.int32, (MD, M), 1)
    ohA = (adeg == col).astype(jnp.float32)
    ohB = (bdeg == col).astype(jnp.float32)

    attend_mask = (adeg != M - 1).astype(jnp.float32).reshape(M, D, 1)
    smask = (adeg == M - 1).astype(jnp.float32).reshape(M, D, 1) * -9e8

    atom_feat = _lrelu(_dot(atom, Wa_t[...]) + ba[...])
    nf = _lrelu(_dot(ohA, _dot(atom, WnA_t[...]))
                + _dot(ohB, _dot(bond, WnB_t[...])) + bn[...])

    h = atom_feat
    feat = atom_feat
    for r in range(RADIUS):
        if r == 0:
            att = _dot(nf, Wt_t[0]) + bt[0]
            s_n = jnp.sum(nf * wal_n[0], axis=1, keepdims=True)
        else:
            af = jnp.maximum(h, 0.0)
            feat = af
            att = _dot(ohA, _dot(af, Wt_t[r])) + bt[r]
            s_n = _dot(ohA, jnp.sum(af * wal_n[r], axis=1, keepdims=True))
        s_a = jnp.sum(feat * wal_a[r], axis=1, keepdims=True)
        score = _lrelu(s_a[:, None, :] + s_n.reshape(M, D, 1) + bal[r]) + smask
        aw = _softmax(score, axis=1) * attend_mask
        ctx = _elu(jnp.sum(aw * att.reshape(M, D, FP), axis=1))
        x_in = atom_feat if r == 0 else h
        h = _gru(ctx, x_in, Wi_t[r], Wh_t[r], bi[r], bh[r])
        feat = h

    af_out_ref[0] = h
    af_fin = jnp.maximum(h, 0.0)

    mol = jnp.sum(af_fin * amask, axis=0, keepdims=True)
    afm = jnp.maximum(mol, 0.0)
    msm = (amask == 0.0).astype(jnp.float32) * -9e8
    attm = _dot(af_fin, Wmt_t[...]) + bmt[...]
    for _t in range(T_STEPS):
        s_m = jnp.sum(afm * wma_m[...], axis=1, keepdims=True)
        s_x = jnp.sum(af_fin * wma_x[...], axis=1, keepdims=True)
        score = _lrelu(s_m + s_x + bma[...]) + msm
        aw = _softmax(score, axis=0) * amask
        mctx = _elu(jnp.sum(aw * attm, axis=0, keepdims=True))
        mol = _gru(mctx, mol, mWi_t[...], mWh_t[...], mbi[...], mbh[...])
        afm = jnp.maximum(mol, 0.0)

    pred = jnp.sum(mol * wo[...], axis=1, keepdims=True) + bo[...]
    pred_out_ref[...] = jnp.broadcast_to(pred, (1, 1, 128))


def kernel(atom_list, bond_list, atom_degree_list, bond_degree_list, atom_mask, params):
    p = params
    Wa, ba = p['atom_fc']
    Wn, bn = p['neighbor_fc']
    Wma, bma = p['mol_align']
    Wmt, bmt = p['mol_attend']
    mWi, mWh, mbi, mbh = p['mol_gru']
    Wo, bo = p['output']

    wal_a = jnp.stack([p['align'][r][0][0, :FP] for r in range(RADIUS)])[:, None, :]
    wal_n = jnp.stack([p['align'][r][0][0, FP:] for r in range(RADIUS)])[:, None, :]
    bal = jnp.stack([p['align'][r][1] for r in range(RADIUS)])[:, :, None]
    Wt_t = jnp.stack([p['attend'][r][0].T for r in range(RADIUS)])
    bt = jnp.stack([p['attend'][r][1] for r in range(RADIUS)])[:, None, :]
    Wi_t = jnp.stack([p['gru'][r][0].T for r in range(RADIUS)])
    Wh_t = jnp.stack([p['gru'][r][1].T for r in range(RADIUS)])
    bi = jnp.stack([p['gru'][r][2] for r in range(RADIUS)])[:, None, :]
    bh = jnp.stack([p['gru'][r][3] for r in range(RADIUS)])[:, None, :]

    adeg = atom_degree_list.astype(jnp.int32).reshape(B, MD, 1)
    bdeg = bond_degree_list.astype(jnp.int32).reshape(B, MD, 1)
    amask = atom_mask.reshape(B, M, 1)

    data_in = [atom_list, bond_list, adeg, bdeg, amask]
    data_specs = [
        pl.BlockSpec((1, M, FA), lambda i: (i, 0, 0)),
        pl.BlockSpec((1, M, FB), lambda i: (i, 0, 0)),
        pl.BlockSpec((1, MD, 1), lambda i: (i, 0, 0)),
        pl.BlockSpec((1, MD, 1), lambda i: (i, 0, 0)),
        pl.BlockSpec((1, M, 1), lambda i: (i, 0, 0)),
    ]
    param_in = [
        Wa.T, ba[None], Wn[:, :FA].T, Wn[:, FA:].T, bn[None],
        wal_a, wal_n, bal, Wt_t, bt, Wi_t, Wh_t, bi, bh,
        Wma[:, :FP], Wma[:, FP:], bma[None], Wmt.T, bmt[None],
        mWi.T, mWh.T, mbi[None], mbh[None],
        Wo, bo[None],
    ]
    param_specs = [
        pl.BlockSpec(x.shape, functools.partial(lambda n, i: (0,) * n, x.ndim))
        for x in param_in
    ]

    out_shapes = [
        jax.ShapeDtypeStruct((B, M, FP), jnp.float32),
        jax.ShapeDtypeStruct((B, 1, 128), jnp.float32),
    ]
    out_specs = [
        pl.BlockSpec((1, M, FP), lambda i: (i, 0, 0)),
        pl.BlockSpec((1, 1, 128), lambda i: (i, 0, 0)),
    ]

    af, pred = pl.pallas_call(
        _body,
        grid=(B,),
        in_specs=data_specs + param_specs,
        out_specs=out_specs,
        out_shape=out_shapes,
        compiler_params=pltpu.CompilerParams(
            dimension_semantics=("arbitrary",),
        ),
    )(*data_in, *param_in)
    return (af, pred[:, 0, :1])


# speedup vs baseline: 12.8312x; 1.7156x over previous
import functools

import jax
import jax.numpy as jnp
from jax.experimental import pallas as pl
from jax.experimental.pallas import tpu as pltpu

B, M, D = 256, 128, 8
FA, FB, FP = 64, 16, 64
RADIUS, T_STEPS = 3, 2
MD = M * D

_PREC = jax.lax.Precision.DEFAULT


def _dot(a, b):
    return jax.lax.dot(a, b, precision=_PREC, preferred_element_type=jnp.float32)


def _lrelu(x):
    return jnp.where(x >= 0, x, 0.01 * x)


def _elu(x):
    return jnp.where(x > 0, x, jnp.exp(jnp.minimum(x, 0.0)) - 1.0)


def _softmax(x, axis):
    m = jnp.max(x, axis=axis, keepdims=True)
    e = jnp.exp(x - m)
    return e / jnp.sum(e, axis=axis, keepdims=True)


def _gru(x, h, Wi_t, Wh_t, bi, bh):
    gi = _dot(x, Wi_t) + bi
    gh = _dot(h, Wh_t) + bh
    r = jax.nn.sigmoid(gi[:, :FP] + gh[:, :FP])
    z = jax.nn.sigmoid(gi[:, FP:2 * FP] + gh[:, FP:2 * FP])
    n = jnp.tanh(gi[:, 2 * FP:] + r * gh[:, 2 * FP:])
    return (1.0 - z) * n + z * h


def _body(atom_ref, bond_ref, adeg_ref, bdeg_ref, amask_ref,
          Wa_t, ba, WnA_t, WnB_t, bn,
          wal_a, wal_n, bal, Wt_t, bt, Wi_t, Wh_t, bi, bh,
          wma_m, wma_x, bma, Wmt_t, bmt, mWi_t, mWh_t, mbi, mbh,
          wo, bo,
          af_out_ref, pred_out_ref):
    atom = atom_ref[0]
    bond = bond_ref[0]
    adeg = adeg_ref[0]
    bdeg = bdeg_ref[0]
    amask = amask_ref[0]

    col = jax.lax.broadcasted_iota(jnp.int32, (MD, M), 1)
    ohA = (adeg == col).astype(jnp.float32)
    ohB = (bdeg == col).astype(jnp.float32)

    attend_mask = (adeg != M - 1).astype(jnp.float32).reshape(M, D, 1)
    smask = (adeg == M - 1).astype(jnp.float32).reshape(M, D, 1) * -9e8

    atom_feat = _lrelu(_dot(atom, Wa_t[...]) + ba[...])
    nf = _lrelu(_dot(ohA, _dot(atom, WnA_t[...]))
                + _dot(ohB, _dot(bond, WnB_t[...])) + bn[...])

    h = atom_feat
    feat = atom_feat
    for r in range(RADIUS):
        if r == 0:
            att = _dot(nf, Wt_t[0]) + bt[0]
            s_n = jnp.sum(nf * wal_n[0], axis=1, keepdims=True)
        else:
            af = jnp.maximum(h, 0.0)
            feat = af
            att = _dot(ohA, _dot(af, Wt_t[r])) + bt[r]
            s_n = _dot(ohA, jnp.sum(af * wal_n[r], axis=1, keepdims=True))
        s_a = jnp.sum(feat * wal_a[r], axis=1, keepdims=True)
        score = _lrelu(s_a[:, None, :] + s_n.reshape(M, D, 1) + bal[r]) + smask
        aw = _softmax(score, axis=1) * attend_mask
        ctx = _elu(jnp.sum(aw * att.reshape(M, D, FP), axis=1))
        x_in = atom_feat if r == 0 else h
        h = _gru(ctx, x_in, Wi_t[r], Wh_t[r], bi[r], bh[r])
        feat = h

    af_out_ref[0] = h
    af_fin = jnp.maximum(h, 0.0)

    mol = jnp.sum(af_fin * amask, axis=0, keepdims=True)
    afm = jnp.maximum(mol, 0.0)
    msm = (amask == 0.0).astype(jnp.float32) * -9e8
    attm = _dot(af_fin, Wmt_t[...]) + bmt[...]
    for _t in range(T_STEPS):
        s_m = jnp.sum(afm * wma_m[...], axis=1, keepdims=True)
        s_x = jnp.sum(af_fin * wma_x[...], axis=1, keepdims=True)
        score = _lrelu(s_m + s_x + bma[...]) + msm
        aw = _softmax(score, axis=0) * amask
        mctx = _elu(jnp.sum(aw * attm, axis=0, keepdims=True))
        mol = _gru(mctx, mol, mWi_t[...], mWh_t[...], mbi[...], mbh[...])
        afm = jnp.maximum(mol, 0.0)

    pred = jnp.sum(mol * wo[...], axis=1, keepdims=True) + bo[...]
    pred_out_ref[...] = jnp.broadcast_to(pred, (1, 1, 128))


def kernel(atom_list, bond_list, atom_degree_list, bond_degree_list, atom_mask, params):
    p = params
    Wa, ba = p['atom_fc']
    Wn, bn = p['neighbor_fc']
    Wma, bma = p['mol_align']
    Wmt, bmt = p['mol_attend']
    mWi, mWh, mbi, mbh = p['mol_gru']
    Wo, bo = p['output']

    wal_a = jnp.stack([p['align'][r][0][0, :FP] for r in range(RADIUS)])[:, None, :]
    wal_n = jnp.stack([p['align'][r][0][0, FP:] for r in range(RADIUS)])[:, None, :]
    bal = jnp.stack([p['align'][r][1] for r in range(RADIUS)])[:, :, None]
    Wt_t = jnp.stack([p['attend'][r][0].T for r in range(RADIUS)])
    bt = jnp.stack([p['attend'][r][1] for r in range(RADIUS)])[:, None, :]
    Wi_t = jnp.stack([p['gru'][r][0].T for r in range(RADIUS)])
    Wh_t = jnp.stack([p['gru'][r][1].T for r in range(RADIUS)])
    bi = jnp.stack([p['gru'][r][2] for r in range(RADIUS)])[:, None, :]
    bh = jnp.stack([p['gru'][r][3] for r in range(RADIUS)])[:, None, :]

    adeg = atom_degree_list.astype(jnp.int32).reshape(B, MD, 1)
    bdeg = bond_degree_list.astype(jnp.int32).reshape(B, MD, 1)
    amask = atom_mask.reshape(B, M, 1)

    data_in = [atom_list, bond_list, adeg, bdeg, amask]
    data_specs = [
        pl.BlockSpec((1, M, FA), lambda i: (i, 0, 0)),
        pl.BlockSpec((1, M, FB), lambda i: (i, 0, 0)),
        pl.BlockSpec((1, MD, 1), lambda i: (i, 0, 0)),
        pl.BlockSpec((1, MD, 1), lambda i: (i, 0, 0)),
        pl.BlockSpec((1, M, 1), lambda i: (i, 0, 0)),
    ]
    param_in = [
        Wa.T, ba[None], Wn[:, :FA].T, Wn[:, FA:].T, bn[None],
        wal_a, wal_n, bal, Wt_t, bt, Wi_t, Wh_t, bi, bh,
        Wma[:, :FP], Wma[:, FP:], bma[None], Wmt.T, bmt[None],
        mWi.T, mWh.T, mbi[None], mbh[None],
        Wo, bo[None],
    ]
    param_specs = [
        pl.BlockSpec(x.shape, functools.partial(lambda n, i: (0,) * n, x.ndim))
        for x in param_in
    ]

    out_shapes = [
        jax.ShapeDtypeStruct((B, M, FP), jnp.float32),
        jax.ShapeDtypeStruct((B, 1, 128), jnp.float32),
    ]
    out_specs = [
        pl.BlockSpec((1, M, FP), lambda i: (i, 0, 0)),
        pl.BlockSpec((1, 1, 128), lambda i: (i, 0, 0)),
    ]

    af, pred = pl.pallas_call(
        _body,
        grid=(B,),
        in_specs=data_specs + param_specs,
        out_specs=out_specs,
        out_shape=out_shapes,
        compiler_params=pltpu.CompilerParams(
            dimension_semantics=("arbitrary",),
        ),
    )(*data_in, *param_in)
    return (af, pred[:, 0, :1])


# speedup vs baseline: 13.0497x; 1.0170x over previous
import functools

import jax
import jax.numpy as jnp
from jax.experimental import pallas as pl
from jax.experimental.pallas import tpu as pltpu

B, M, D = 256, 128, 8
FA, FB, FP = 64, 16, 64
RADIUS, T_STEPS = 3, 2
MD = M * D

_PREC = jax.lax.Precision.DEFAULT


def _dot(a, b):
    return jax.lax.dot(a, b, precision=_PREC, preferred_element_type=jnp.float32)


def _lrelu(x):
    return jnp.where(x >= 0, x, 0.01 * x)


def _elu(x):
    return jnp.where(x > 0, x, jnp.exp(jnp.minimum(x, 0.0)) - 1.0)


def _gru(x, h, Wi_t, Wh_t, bi, bh):
    gi = _dot(x, Wi_t) + bi
    gh = _dot(h, Wh_t) + bh
    r = jax.nn.sigmoid(gi[:, :FP] + gh[:, :FP])
    z = jax.nn.sigmoid(gi[:, FP:2 * FP] + gh[:, FP:2 * FP])
    n = jnp.tanh(gi[:, 2 * FP:] + r * gh[:, 2 * FP:])
    return (1.0 - z) * n + z * h


def _body(atom_ref, bond_ref, adeg_ref, bdeg_ref, amask_ref,
          Wa_t, ba, WnA_t, WnB_t, bn,
          cat0, catR, bal, bt, Wi_t, Wh_t, bi, bh,
          wma_x_rep, wma_m_rep, bma, Wmt_t, bmt, mWi_t, mWh_t, mbi, mbh,
          wo, bo,
          af_out_ref, pred_out_ref):
    atom = atom_ref[0]
    bond = bond_ref[0]
    adeg = adeg_ref[0]
    bdeg = bdeg_ref[0]
    amask = amask_ref[0]

    col = jax.lax.broadcasted_iota(jnp.int32, (MD, M), 1)
    ohA = (adeg == col).astype(jnp.float32)
    ohB = (bdeg == col).astype(jnp.float32)
    seg = jax.lax.broadcasted_iota(jnp.int32, (M, MD), 1) // D
    S = (seg == jax.lax.broadcasted_iota(jnp.int32, (M, MD), 0)).astype(jnp.float32)

    adeg64 = jnp.broadcast_to(adeg, (MD, FP))
    smask = jnp.where(adeg64 == M - 1, -9e8, 0.0)

    atom_feat = _lrelu(_dot(atom, Wa_t[...]) + ba[...])
    nf = _lrelu(_dot(ohA, _dot(atom, WnA_t[...]))
                + _dot(ohB, _dot(bond, WnB_t[...])) + bn[...])

    h = atom_feat
    for r in range(RADIUS):
        if r == 0:
            gath = _dot(nf, cat0[...])
            sa = _dot(atom_feat, catR[0, :, 2 * FP:])
        else:
            af = jnp.maximum(h, 0.0)
            af_ext = _dot(af, catR[r])
            gath = _dot(ohA, af_ext[:, :2 * FP])
            sa = af_ext[:, 2 * FP:]
        att = gath[:, :FP] + bt[r]
        sn = gath[:, FP:]
        score = (sn.reshape(M, D, FP) + sa[:, None, :]).reshape(MD, FP)
        score = _lrelu(score + bal[r]) + smask
        e = jnp.exp(score)
        nd = _dot(S, jnp.concatenate([e * att, e], axis=1))
        ctx = _elu(nd[:, :FP] / jnp.maximum(nd[:, FP:], 1e-30))
        x_h = atom_feat if r == 0 else h
        h = _gru(ctx, x_h, Wi_t[r], Wh_t[r], bi[r], bh[r])

    af_out_ref[0] = h
    af_fin = jnp.maximum(h, 0.0)

    amask64 = jnp.broadcast_to(amask, (M, FP))
    msm = jnp.where(amask64 == 0.0, -9e8, 0.0)
    ones_1m = jnp.ones((1, M), jnp.float32)
    mol = _dot(ones_1m, af_fin * amask64)
    afm = jnp.maximum(mol, 0.0)
    attm = _dot(af_fin, Wmt_t[...]) + bmt[...]
    sx = _dot(af_fin, wma_x_rep[...])
    for _t in range(T_STEPS):
        sm = _dot(afm, wma_m_rep[...])
        score = _lrelu(sx + sm + bma[...]) + msm
        e = jnp.exp(score)
        nd = _dot(ones_1m, jnp.concatenate([e * attm, e], axis=1))
        mctx = _elu(nd[:, :FP] / jnp.maximum(nd[:, FP:], 1e-30))
        mol = _gru(mctx, mol, mWi_t[...], mWh_t[...], mbi[...], mbh[...])
        afm = jnp.maximum(mol, 0.0)

    pred = jnp.sum(mol * wo[...], axis=1, keepdims=True) + bo[...]
    pred_out_ref[...] = jnp.broadcast_to(pred, (1, 1, 128))


def kernel(atom_list, bond_list, atom_degree_list, bond_degree_list, atom_mask, params):
    p = params
    Wa, ba = p['atom_fc']
    Wn, bn = p['neighbor_fc']
    Wma, bma = p['mol_align']
    Wmt, bmt = p['mol_attend']
    mWi, mWh, mbi, mbh = p['mol_gru']
    Wo, bo = p['output']

    def rep(v):
        return jnp.broadcast_to(v[:, None], (FP, FP))

    cat0 = jnp.concatenate([p['attend'][0][0].T, rep(p['align'][0][0][0, FP:])], axis=1)
    catR = jnp.stack([
        jnp.concatenate([p['attend'][r][0].T,
                         rep(p['align'][r][0][0, FP:]),
                         rep(p['align'][r][0][0, :FP])], axis=1)
        for r in range(RADIUS)])
    bal = jnp.stack([p['align'][r][1] for r in range(RADIUS)])[:, :, None]
    bt = jnp.stack([p['attend'][r][1] for r in range(RADIUS)])[:, None, :]
    Wi_t = jnp.stack([p['gru'][r][0].T for r in range(RADIUS)])
    Wh_t = jnp.stack([p['gru'][r][1].T for r in range(RADIUS)])
    bi = jnp.stack([p['gru'][r][2] for r in range(RADIUS)])[:, None, :]
    bh = jnp.stack([p['gru'][r][3] for r in range(RADIUS)])[:, None, :]

    adeg = atom_degree_list.astype(jnp.int32).reshape(B, MD, 1)
    bdeg = bond_degree_list.astype(jnp.int32).reshape(B, MD, 1)
    amask = atom_mask.reshape(B, M, 1)

    data_in = [atom_list, bond_list, adeg, bdeg, amask]
    data_specs = [
        pl.BlockSpec((1, M, FA), lambda i: (i, 0, 0)),
        pl.BlockSpec((1, M, FB), lambda i: (i, 0, 0)),
        pl.BlockSpec((1, MD, 1), lambda i: (i, 0, 0)),
        pl.BlockSpec((1, MD, 1), lambda i: (i, 0, 0)),
        pl.BlockSpec((1, M, 1), lambda i: (i, 0, 0)),
    ]
    param_in = [
        Wa.T, ba[None], Wn[:, :FA].T, Wn[:, FA:].T, bn[None],
        cat0, catR, bal, bt, Wi_t, Wh_t, bi, bh,
        rep(Wma[0, FP:]), rep(Wma[0, :FP]), bma[None], Wmt.T, bmt[None],
        mWi.T, mWh.T, mbi[None], mbh[None],
        Wo, bo[None],
    ]
    param_specs = [
        pl.BlockSpec(x.shape, functools.partial(lambda n, i: (0,) * n, x.ndim))
        for x in param_in
    ]

    out_shapes = [
        jax.ShapeDtypeStruct((B, M, FP), jnp.float32),
        jax.ShapeDtypeStruct((B, 1, 128), jnp.float32),
    ]
    out_specs = [
        pl.BlockSpec((1, M, FP), lambda i: (i, 0, 0)),
        pl.BlockSpec((1, 1, 128), lambda i: (i, 0, 0)),
    ]

    af, pred = pl.pallas_call(
        _body,
        grid=(B,),
        in_specs=data_specs + param_specs,
        out_specs=out_specs,
        out_shape=out_shapes,
        compiler_params=pltpu.CompilerParams(
            dimension_semantics=("arbitrary",),
        ),
    )(*data_in, *param_in)
    return (af, pred[:, 0, :1])


# speedup vs baseline: 23.6317x; 1.8109x over previous
import functools

import jax
import jax.numpy as jnp
from jax.experimental import pallas as pl
from jax.experimental.pallas import tpu as pltpu

B, M, D = 256, 128, 8
FA, FB, FP = 64, 16, 64
RADIUS, T_STEPS = 3, 2
MD = M * D
BM = 4

_PREC = jax.lax.Precision.DEFAULT


def _dot(a, b):
    return jax.lax.dot(a, b, precision=_PREC, preferred_element_type=jnp.float32)


def _lrelu(x):
    return jnp.where(x >= 0, x, 0.01 * x)


def _elu(x):
    return jnp.where(x > 0, x, jnp.exp(jnp.minimum(x, 0.0)) - 1.0)


def _gru(x, h, Wi_t, Wh_t, bi, bh):
    gi = _dot(x, Wi_t) + bi
    gh = _dot(h, Wh_t) + bh
    r = jax.nn.sigmoid(gi[:, :FP] + gh[:, :FP])
    z = jax.nn.sigmoid(gi[:, FP:2 * FP] + gh[:, FP:2 * FP])
    n = jnp.tanh(gi[:, 2 * FP:] + r * gh[:, 2 * FP:])
    return (1.0 - z) * n + z * h


def _body(atom_ref, bond_ref, adeg_ref, bdeg_ref, amask_ref,
          Wa_t, ba, WnA_t, WnB_t, bn,
          cat0, catR, bal, bt, Wi_t, Wh_t, bi, bh,
          wma_x_rep, wma_m_rep, bma, Wmt_t, bmt, mWi_t, mWh_t, mbi, mbh,
          wo, bo,
          af_out_ref, pred_out_ref):
    atom = atom_ref[...].reshape(BM * M, FA)
    bond = bond_ref[...].reshape(BM * M, FB)
    adeg = adeg_ref[...]
    bdeg = bdeg_ref[...]
    amask = amask_ref[...].reshape(BM * M, 1)

    col = jax.lax.broadcasted_iota(jnp.int32, (MD, M), 1)
    ohA = [(adeg[i] == col).astype(jnp.float32) for i in range(BM)]
    ohB = [(bdeg[i] == col).astype(jnp.float32) for i in range(BM)]
    seg = jax.lax.broadcasted_iota(jnp.int32, (M, MD), 1) // D
    S = (seg == jax.lax.broadcasted_iota(jnp.int32, (M, MD), 0)).astype(jnp.float32)

    adeg_f = adeg.reshape(BM * MD, 1)
    smask = jnp.where(jnp.broadcast_to(adeg_f, (BM * MD, FP)) == M - 1, -9e8, 0.0)

    atom_feat = _lrelu(_dot(atom, Wa_t[...]) + ba[...])
    P = _dot(atom, WnA_t[...])
    Q = _dot(bond, WnB_t[...])
    nf = _lrelu(jnp.concatenate(
        [_dot(ohA[i], P[i * M:(i + 1) * M]) + _dot(ohB[i], Q[i * M:(i + 1) * M])
         for i in range(BM)], axis=0) + bn[...])

    h = atom_feat
    for r in range(RADIUS):
        if r == 0:
            gath = _dot(nf, cat0[...])
            sa = _dot(atom_feat, catR[0, :, 2 * FP:])
        else:
            af = jnp.maximum(h, 0.0)
            af_ext = _dot(af, catR[r])
            gath = jnp.concatenate(
                [_dot(ohA[i], af_ext[i * M:(i + 1) * M, :2 * FP]) for i in range(BM)],
                axis=0)
            sa = af_ext[:, 2 * FP:]
        att = gath[:, :FP] + bt[r]
        sn = gath[:, FP:]
        score = (sn.reshape(BM * M, D, FP) + sa[:, None, :]).reshape(BM * MD, FP)
        score = _lrelu(score + bal[r]) + smask
        e = jnp.exp(score)
        X = jnp.concatenate([e * att, e], axis=1)
        nd = jnp.concatenate(
            [_dot(S, X[i * MD:(i + 1) * MD]) for i in range(BM)], axis=0)
        ctx = _elu(nd[:, :FP] / jnp.maximum(nd[:, FP:], 1e-30))
        x_h = atom_feat if r == 0 else h
        h = _gru(ctx, x_h, Wi_t[r], Wh_t[r], bi[r], bh[r])

    af_out_ref[...] = h.reshape(BM, M, FP)
    af_fin = jnp.maximum(h, 0.0)

    amask64 = jnp.broadcast_to(amask, (BM * M, FP))
    msm = jnp.where(amask64 == 0.0, -9e8, 0.0)
    mseg = jax.lax.broadcasted_iota(jnp.int32, (BM, BM * M), 1) // M
    onesb = (mseg == jax.lax.broadcasted_iota(jnp.int32, (BM, BM * M), 0)).astype(jnp.float32)

    mol = _dot(onesb, af_fin * amask64)
    afm = jnp.maximum(mol, 0.0)
    attm = _dot(af_fin, Wmt_t[...]) + bmt[...]
    sx = _dot(af_fin, wma_x_rep[...])
    for _t in range(T_STEPS):
        sm = _dot(afm, wma_m_rep[...])
        score = (sx.reshape(BM, M, FP) + sm[:, None, :]).reshape(BM * M, FP)
        score = _lrelu(score + bma[...]) + msm
        e = jnp.exp(score)
        nd = _dot(onesb, jnp.concatenate([e * attm, e], axis=1))
        mctx = _elu(nd[:, :FP] / jnp.maximum(nd[:, FP:], 1e-30))
        mol = _gru(mctx, mol, mWi_t[...], mWh_t[...], mbi[...], mbh[...])
        afm = jnp.maximum(mol, 0.0)

    pred = jnp.sum(mol * wo[...], axis=1, keepdims=True) + bo[...]
    pred_out_ref[...] = jnp.broadcast_to(pred.reshape(1, BM, 1), (1, BM, 128))


def kernel(atom_list, bond_list, atom_degree_list, bond_degree_list, atom_mask, params):
    p = params
    Wa, ba = p['atom_fc']
    Wn, bn = p['neighbor_fc']
    Wma, bma = p['mol_align']
    Wmt, bmt = p['mol_attend']
    mWi, mWh, mbi, mbh = p['mol_gru']
    Wo, bo = p['output']

    def rep(v):
        return jnp.broadcast_to(v[:, None], (FP, FP))

    cat0 = jnp.concatenate([p['attend'][0][0].T, rep(p['align'][0][0][0, FP:])], axis=1)
    catR = jnp.stack([
        jnp.concatenate([p['attend'][r][0].T,
                         rep(p['align'][r][0][0, FP:]),
                         rep(p['align'][r][0][0, :FP])], axis=1)
        for r in range(RADIUS)])
    bal = jnp.stack([p['align'][r][1] for r in range(RADIUS)])[:, :, None]
    bt = jnp.stack([p['attend'][r][1] for r in range(RADIUS)])[:, None, :]
    Wi_t = jnp.stack([p['gru'][r][0].T for r in range(RADIUS)])
    Wh_t = jnp.stack([p['gru'][r][1].T for r in range(RADIUS)])
    bi = jnp.stack([p['gru'][r][2] for r in range(RADIUS)])[:, None, :]
    bh = jnp.stack([p['gru'][r][3] for r in range(RADIUS)])[:, None, :]

    adeg = atom_degree_list.astype(jnp.int32).reshape(B, MD, 1)
    bdeg = bond_degree_list.astype(jnp.int32).reshape(B, MD, 1)
    amask = atom_mask.reshape(B, M, 1)

    data_in = [atom_list, bond_list, adeg, bdeg, amask]
    data_specs = [
        pl.BlockSpec((BM, M, FA), lambda i: (i, 0, 0)),
        pl.BlockSpec((BM, M, FB), lambda i: (i, 0, 0)),
        pl.BlockSpec((BM, MD, 1), lambda i: (i, 0, 0)),
        pl.BlockSpec((BM, MD, 1), lambda i: (i, 0, 0)),
        pl.BlockSpec((BM, M, 1), lambda i: (i, 0, 0)),
    ]
    param_in = [
        Wa.T, ba[None], Wn[:, :FA].T, Wn[:, FA:].T, bn[None],
        cat0, catR, bal, bt, Wi_t, Wh_t, bi, bh,
        rep(Wma[0, FP:]), rep(Wma[0, :FP]), bma[None], Wmt.T, bmt[None],
        mWi.T, mWh.T, mbi[None], mbh[None],
        Wo, bo[None],
    ]
    param_specs = [
        pl.BlockSpec(x.shape, functools.partial(lambda n, i: (0,) * n, x.ndim))
        for x in param_in
    ]

    out_shapes = [
        jax.ShapeDtypeStruct((B, M, FP), jnp.float32),
        jax.ShapeDtypeStruct((B // BM, BM, 128), jnp.float32),
    ]
    out_specs = [
        pl.BlockSpec((BM, M, FP), lambda i: (i, 0, 0)),
        pl.BlockSpec((1, BM, 128), lambda i: (i, 0, 0)),
    ]

    af, pred = pl.pallas_call(
        _body,
        grid=(B // BM,),
        in_specs=data_specs + param_specs,
        out_specs=out_specs,
        out_shape=out_shapes,
        compiler_params=pltpu.CompilerParams(
            dimension_semantics=("arbitrary",),
        ),
    )(*data_in, *param_in)
    return (af, pred.reshape(B, 128)[:, :1])
